# baseline (speedup 1.0000x reference)
import functools

import jax
import jax.numpy as jnp
from jax import lax
from jax.experimental import pallas as pl
from jax.experimental.pallas import tpu as pltpu
from jax.experimental.pallas import tpu_sc as plsc

MODEL_DIM = 64
SCALE = 8.0

NUM_CORES = 2
NUM_SUBCORES = 16
NUM_WORKERS = NUM_CORES * NUM_SUBCORES

BATCH = 4096
SEQ = 200
LANES = 16
UNIT = 128
N_UNITS = BATCH * SEQ // UNIT
UNITS_PER_WORKER = N_UNITS // NUM_WORKERS
IDX_PER_WORKER = UNITS_PER_WORKER * UNIT
TSTRIDE = 137


@functools.partial(
    pl.kernel,
    out_type=jax.ShapeDtypeStruct((SEQ, MODEL_DIM, BATCH), jnp.float32),
    mesh=plsc.VectorSubcoreMesh(core_axis_name="c", subcore_axis_name="s"),
    compiler_params=pltpu.CompilerParams(needs_layout_passes=False),
    scratch_types=[
        pltpu.VMEM((IDX_PER_WORKER,), jnp.int32),
        pltpu.VMEM((UNIT, 128), jnp.float32),
        pltpu.VMEM((UNIT, 128), jnp.float32),
        pltpu.VMEM((MODEL_DIM, TSTRIDE), jnp.float32),
        pltpu.VMEM((MODEL_DIM, TSTRIDE), jnp.float32),
        pltpu.SemaphoreType.DMA,
        pltpu.SemaphoreType.DMA,
        pltpu.SemaphoreType.DMA,
        pltpu.SemaphoreType.DMA,
    ],
)
def _emb_lookup(table_hbm, idx_hbm, out_hbm, idx_v, buf0, buf1, tm0, tm1,
                gsem0, gsem1, wsem0, wsem1):
    wid = lax.axis_index("s") * NUM_CORES + lax.axis_index("c")
    ubase = wid * UNITS_PER_WORKER
    pltpu.sync_copy(idx_hbm.at[pl.ds(ubase * UNIT, IDX_PER_WORKER)], idx_v)

    rows = [lax.iota(jnp.int32, LANES) + c * LANES
            for c in range(MODEL_DIM // LANES)]

    def gather(i, buf, sem):
        pltpu.async_copy(table_hbm.at[idx_v.at[pl.ds(i * UNIT, UNIT)]],
                         buf, sem)

    def wait_gather(buf, sem):
        pltpu.make_async_copy(table_hbm.at[idx_v.at[pl.ds(0, UNIT)]],
                              buf, sem).wait()

    def transpose_scale(buf, tm):
        @plsc.parallel_loop(0, UNIT, unroll=4)
        def _(b):
            col = jnp.full((LANES,), b, dtype=jnp.int32)
            for c in range(MODEL_DIM // LANES):
                v = buf[b, pl.ds(c * LANES, LANES)]
                plsc.store_scatter(tm, [rows[c], col], v * SCALE)

    def writeback(i, tm, sem):
        u = ubase + i
        t = u // (BATCH // UNIT)
        bg = lax.rem(u, BATCH // UNIT)
        pltpu.async_copy(tm.at[:, pl.ds(0, UNIT)],
                         out_hbm.at[t, :, pl.ds(bg * UNIT, UNIT)], sem)

    def wait_writeback(tm, sem):
        pltpu.make_async_copy(tm.at[:, pl.ds(0, UNIT)],
                              out_hbm.at[0, :, pl.ds(0, UNIT)], sem).wait()

    gather(0, buf0, gsem0)
    gather(1, buf1, gsem1)

    def half(i, u, buf, gsem, tm, wsem):
        wait_gather(buf, gsem)

        @pl.when(i > 0)
        def _():
            wait_writeback(tm, wsem)
        transpose_scale(buf, tm)

        @pl.when(u + 2 < UNITS_PER_WORKER)
        def _():
            gather(u + 2, buf, gsem)
        writeback(u, tm, wsem)

    def body(i, _):
        i0 = 2 * i
        half(i, i0, buf0, gsem0, tm0, wsem0)
        half(i, i0 + 1, buf1, gsem1, tm1, wsem1)
        return 0

    lax.fori_loop(0, UNITS_PER_WORKER // 2, body, 0)
    wait_writeback(tm0, wsem0)
    wait_writeback(tm1, wsem1)


def kernel(inputs, embeddings):
    table128 = jnp.pad(embeddings, ((0, 0), (0, 128 - MODEL_DIM)))
    idx = inputs.T.reshape(-1)
    out3 = _emb_lookup(table128, idx)
    return out3.transpose(2, 0, 1)


# speedup vs baseline: 1.5725x; 1.5725x over previous
import functools

import jax
import jax.numpy as jnp
from jax import lax
from jax.experimental import pallas as pl
from jax.experimental.pallas import tpu as pltpu
from jax.experimental.pallas import tpu_sc as plsc

MODEL_DIM = 64
SCALE = 8.0

NUM_CORES = 2
NUM_SUBCORES = 16
NUM_WORKERS = NUM_CORES * NUM_SUBCORES

BATCH = 4096
SEQ = 200
LANES = 16
UNIT = 128
N_UNITS = BATCH * SEQ // UNIT
UNITS_PER_WORKER = N_UNITS // NUM_WORKERS
IDX_PER_WORKER = UNITS_PER_WORKER * UNIT
TSTRIDE = 137


@functools.partial(
    pl.kernel,
    out_type=jax.ShapeDtypeStruct((SEQ, MODEL_DIM, BATCH), jnp.float32),
    mesh=plsc.VectorSubcoreMesh(core_axis_name="c", subcore_axis_name="s"),
    compiler_params=pltpu.CompilerParams(needs_layout_passes=False),
    scratch_types=[
        pltpu.VMEM((IDX_PER_WORKER,), jnp.int32),
        pltpu.VMEM((UNIT, 128), jnp.float32),
        pltpu.VMEM((UNIT, 128), jnp.float32),
        pltpu.VMEM((MODEL_DIM * TSTRIDE,), jnp.float32),
        pltpu.VMEM((MODEL_DIM, 128), jnp.float32),
        pltpu.VMEM((MODEL_DIM, 128), jnp.float32),
        pltpu.SemaphoreType.DMA,
        pltpu.SemaphoreType.DMA,
        pltpu.SemaphoreType.DMA,
        pltpu.SemaphoreType.DMA,
    ],
)
def _emb_lookup(table_hbm, idx_hbm, out_hbm, idx_v, buf0, buf1, tmp, tb0, tb1,
                gsem0, gsem1, wsem0, wsem1):
    wid = lax.axis_index("s") * NUM_CORES + lax.axis_index("c")
    ubase = wid * UNITS_PER_WORKER
    pltpu.sync_copy(idx_hbm.at[pl.ds(ubase * UNIT, IDX_PER_WORKER)], idx_v)

    scat = [(lax.iota(jnp.int32, LANES) + c * LANES) * TSTRIDE
            for c in range(MODEL_DIM // LANES)]

    def gather(i, buf, sem):
        pltpu.async_copy(table_hbm.at[idx_v.at[pl.ds(i * UNIT, UNIT)]],
                         buf, sem)

    def wait_gather(buf, sem):
        pltpu.make_async_copy(table_hbm.at[idx_v.at[pl.ds(0, UNIT)]],
                              buf, sem).wait()

    def transpose_scale(buf, tb):
        @plsc.parallel_loop(0, UNIT, unroll=8)
        def _(b):
            off = jnp.full((LANES,), b, dtype=jnp.int32)
            for c in range(MODEL_DIM // LANES):
                v = buf[b, pl.ds(c * LANES, LANES)]
                plsc.store_scatter(tmp, [scat[c] + off], v * SCALE)

        @plsc.parallel_loop(0, MODEL_DIM, unroll=4)
        def _(d):
            base = d * TSTRIDE
            for c in range(128 // LANES):
                tb[d, pl.ds(c * LANES, LANES)] = (
                    tmp[pl.ds(base + c * LANES, LANES)])

    def writeback(i, tb, sem):
        u = ubase + i
        t = u // (BATCH // UNIT)
        bg = lax.rem(u, BATCH // UNIT)
        pltpu.async_copy(tb, out_hbm.at[t, :, pl.ds(bg * UNIT, UNIT)], sem)

    def wait_writeback(tb, sem):
        pltpu.make_async_copy(tb, out_hbm.at[0, :, pl.ds(0, UNIT)], sem).wait()

    gather(0, buf0, gsem0)
    gather(1, buf1, gsem1)

    def half(i, u, buf, gsem, tb, wsem):
        wait_gather(buf, gsem)

        @pl.when(i > 0)
        def _():
            wait_writeback(tb, wsem)
        transpose_scale(buf, tb)

        @pl.when(u + 2 < UNITS_PER_WORKER)
        def _():
            gather(u + 2, buf, gsem)
        writeback(u, tb, wsem)

    def body(i, _):
        i0 = 2 * i
        half(i, i0, buf0, gsem0, tb0, wsem0)
        half(i, i0 + 1, buf1, gsem1, tb1, wsem1)
        return 0

    lax.fori_loop(0, UNITS_PER_WORKER // 2, body, 0)
    wait_writeback(tb0, wsem0)
    wait_writeback(tb1, wsem1)


def kernel(inputs, embeddings):
    table128 = jnp.concatenate(
        [embeddings,
         jnp.zeros((embeddings.shape[0], 128 - MODEL_DIM), jnp.float32)],
        axis=1)
    idx = inputs.T.reshape(-1)
    out3 = _emb_lookup(table128, idx)
    return out3.transpose(2, 0, 1)
